# Initial kernel scaffold; baseline (speedup 1.0000x reference)
import jax
import jax.numpy as jnp
from jax.experimental import pallas as pl


def kernel(x, pos, batch, edge_index_3rd, W_emb, b_emb, W_filt, b_filt, W_upd, b_upd, W_lin1, b_lin1, Wd1, bd1, Wd2, bd2, Wd3, bd3, Wn1, bn1, Wn2, bn2):
    raise NotImplementedError("write your pallas kernel here")


# speedup vs baseline: 2.1370x; 2.1370x over previous
import functools

import numpy as np
import jax
import jax.numpy as jnp
from jax import lax
from jax.experimental import pallas as pl
from jax.experimental.pallas import tpu as pltpu
from jax.experimental.pallas import tpu_sc as plsc

N = 50000
E = 800000
B = 500
F = 5
HID = 64
LAT = 64
MAXN = 150
CUT = 10.0
NI = 3
NG_D, NG_T, NG_P = 50, 6, 12

_NC = 2
_NS = 16
_NW = _NC * _NS

_RHALF = N // _NC
_RACC_E = 25024
_TRASH_E = 25008
_ZSL_E = _RACC_E // _NS

_BHALF = B // _NC
_RACC_B = 256
_TRASH_B = 252
_ZSL_B = _RACC_B // _NS
_NPAD = 50176


def _sc_gather(table, idx, chunk):
    M = idx.shape[0]
    D = table.shape[1]
    m = M // _NW
    steps = m // chunk
    mesh = plsc.VectorSubcoreMesh(core_axis_name="c", subcore_axis_name="s")

    @functools.partial(
        pl.kernel,
        mesh=mesh,
        out_type=jax.ShapeDtypeStruct((M, D), jnp.float32),
        compiler_params=pltpu.CompilerParams(use_tc_tiling_on_sc=False),
        scratch_types=[
            pltpu.VMEM((chunk,), jnp.int32),
            pltpu.VMEM((chunk, D), jnp.float32),
            pltpu.SemaphoreType.DMA,
        ],
    )
    def k(table_h, idx_h, out_h, idx_v, rows_v, sem):
        wid = lax.axis_index("s") * _NC + lax.axis_index("c")
        base = wid * m

        def body(t, carry):
            off = base + t * chunk
            pltpu.sync_copy(idx_h.at[pl.ds(off, chunk)], idx_v)
            pltpu.async_copy(table_h.at[idx_v], rows_v, sem).wait()
            pltpu.sync_copy(rows_v, out_h.at[pl.ds(off, chunk)])
            return carry

        lax.fori_loop(0, steps, body, 0)

    return k(table, idx)


def _sc_scatter(rows, idx2, zeros, racc, chunk):
    M = rows.shape[0]
    m = M // _NS
    steps = m // chunk
    zsl = racc // _NS
    mesh = plsc.VectorSubcoreMesh(core_axis_name="c", subcore_axis_name="s")

    @functools.partial(
        pl.kernel,
        mesh=mesh,
        out_type=jax.ShapeDtypeStruct((_NC, racc, 64), jnp.float32),
        compiler_params=pltpu.CompilerParams(use_tc_tiling_on_sc=False),
        scratch_types=[
            pltpu.VMEM((chunk,), jnp.int32),
            pltpu.VMEM((chunk, 64), jnp.float32),
            pltpu.VMEM_SHARED((racc, 64), jnp.float32),
            pltpu.SemaphoreType.DMA,
        ],
    )
    def k(rows_h, idx2_h, z_h, out_h, idx_v, rows_v, acc_s, sem):
        c = lax.axis_index("c")
        s = lax.axis_index("s")
        pltpu.sync_copy(z_h, acc_s.at[pl.ds(s * zsl, zsl)])
        plsc.subcore_barrier()

        def body(t, carry):
            off = s * m + t * chunk
            pltpu.sync_copy(rows_h.at[pl.ds(off, chunk)], rows_v)
            pltpu.sync_copy(idx2_h.at[c, pl.ds(off, chunk)], idx_v)
            pltpu.sync_copy(rows_v, acc_s.at[idx_v], add=True)
            return carry

        lax.fori_loop(0, steps, body, 0)
        plsc.subcore_barrier()
        pltpu.sync_copy(acc_s.at[pl.ds(s * zsl, zsl)], out_h.at[c, pl.ds(s * zsl, zsl)])

    return k(rows, idx2, zeros)


def _emb(x, W, b2):
    bm = 5000

    def body(x_ref, w_ref, b_ref, o_ref):
        o_ref[...] = (
            jnp.dot(x_ref[...], w_ref[...], preferred_element_type=jnp.float32)
            + b_ref[...]
        )

    return pl.pallas_call(
        body,
        grid=(N // bm,),
        in_specs=[
            pl.BlockSpec((bm, F), lambda i: (i, 0)),
            pl.BlockSpec((F, HID), lambda i: (0, 0)),
            pl.BlockSpec((1, HID), lambda i: (0, 0)),
        ],
        out_specs=pl.BlockSpec((bm, HID), lambda i: (i, 0)),
        out_shape=jax.ShapeDtypeStruct((N, HID), jnp.float32),
    )(x, W, b2)


def _geom(pg):
    bE = 2000
    nb = E // bE

    def body(pi_ref, pj_ref, pk_ref, pl_ref, o_ref):
        eps = 1e-8

        def c3(r):
            return r[:, 0:1], r[:, 1:2], r[:, 2:3]

        ax, ay, az = c3(pi_ref)
        bx, by, bz = c3(pj_ref)
        cx, cy, cz = c3(pk_ref)
        ex, ey, ez = c3(pl_ref)
        b1x, b1y, b1z = bx - ax, by - ay, bz - az
        b2x, b2y, b2z = cx - bx, cy - by, cz - bz
        b3x, b3y, b3z = ex - cx, ey - cy, ez - cz
        d2 = b1x * b1x + b1y * b1y + b1z * b1z
        dist = jnp.sqrt(d2 + eps)
        nu = jnp.sqrt(d2)
        nv = jnp.sqrt(b2x * b2x + b2y * b2y + b2z * b2z)
        uv = -(b1x * b2x + b1y * b2y + b1z * b2z)
        cos_t = uv / (nu * nv + eps)
        cos_t = jnp.clip(cos_t, -1.0 + 1e-7, 1.0 - 1e-7)
        theta = jnp.arctan2(jnp.sqrt(1.0 - cos_t * cos_t), cos_t)
        n1x = b1y * b2z - b1z * b2y
        n1y = b1z * b2x - b1x * b2z
        n1z = b1x * b2y - b1y * b2x
        n2x = b2y * b3z - b2z * b3y
        n2y = b2z * b3x - b2x * b3z
        n2z = b2x * b3y - b2y * b3x
        crx = n1y * b2z - n1z * b2y
        cry = n1z * b2x - n1x * b2z
        crz = n1x * b2y - n1y * b2x
        y = (crx * n2x + cry * n2y + crz * n2z) / (nv + eps)
        xx = n1x * n2x + n1y * n2y + n1z * n2z + eps
        phi = jnp.arctan2(y, xx)
        C = 0.5 * (jnp.cos(dist * (np.pi / CUT)) + 1.0) * (dist < CUT).astype(jnp.float32)
        pad = jnp.zeros((bE, 4), jnp.float32)
        o_ref[...] = jnp.concatenate([dist, theta, phi, C, pad], axis=1)

    return pl.pallas_call(
        body,
        grid=(nb,),
        in_specs=[
            pl.BlockSpec((bE, 16), lambda i, q=q: (i + q * nb, 0)) for q in range(4)
        ],
        out_specs=pl.BlockSpec((bE, 8), lambda i: (i, 0)),
        out_shape=jax.ShapeDtypeStruct((E, 8), jnp.float32),
    )(pg, pg, pg, pg)


_CD = float(-0.5 / (CUT / (NG_D - 1)) ** 2)
_CT = float(-0.5 / (np.pi / (NG_T - 1)) ** 2)
_CP = float(-0.5 / (2.0 * np.pi / (NG_P - 1)) ** 2)


def _msg(geom, hg, W50, W6, W12, bf, offd, offt, offp):
    bE = 2000
    nb = E // bE

    def body(g_ref, hg_ref, w50_ref, w6_ref, w12_ref, b_ref, od_ref, ot_ref, op_ref, o_ref):
        dist = g_ref[:, 0:1]
        theta = g_ref[:, 1:2]
        phi = g_ref[:, 2:3]
        C = g_ref[:, 3:4]
        rbf = jnp.exp(_CD * (dist - od_ref[...]) ** 2)
        tbf = jnp.exp(_CT * (theta - ot_ref[...]) ** 2)
        pbf = jnp.exp(_CP * (phi - op_ref[...]) ** 2)
        acc = (
            jnp.dot(rbf, w50_ref[...], preferred_element_type=jnp.float32)
            + jnp.dot(tbf, w6_ref[...], preferred_element_type=jnp.float32)
            + jnp.dot(pbf, w12_ref[...], preferred_element_type=jnp.float32)
            + b_ref[...]
        )
        filt = jnp.maximum(acc, 0.0) * C
        o_ref[...] = hg_ref[...] * filt

    return pl.pallas_call(
        body,
        grid=(nb,),
        in_specs=[
            pl.BlockSpec((bE, 8), lambda i: (i, 0)),
            pl.BlockSpec((bE, HID), lambda i: (i, 0)),
            pl.BlockSpec((NG_D, HID), lambda i: (0, 0)),
            pl.BlockSpec((NG_T, HID), lambda i: (0, 0)),
            pl.BlockSpec((NG_P, HID), lambda i: (0, 0)),
            pl.BlockSpec((1, HID), lambda i: (0, 0)),
            pl.BlockSpec((1, NG_D), lambda i: (0, 0)),
            pl.BlockSpec((1, NG_T), lambda i: (0, 0)),
            pl.BlockSpec((1, NG_P), lambda i: (0, 0)),
        ],
        out_specs=pl.BlockSpec((bE, HID), lambda i: (i, 0)),
        out_shape=jax.ShapeDtypeStruct((E, HID), jnp.float32),
    )(geom, hg, W50, W6, W12, bf, offd, offt, offp)


def _upd(h, acc2, Wu, bu):
    bm = 5000
    nb = _RHALF // bm

    def body(h_ref, a_ref, w_ref, b_ref, o_ref):
        a = a_ref[0]
        o_ref[...] = h_ref[...] + jnp.maximum(
            jnp.dot(a, w_ref[...], preferred_element_type=jnp.float32) + b_ref[...],
            0.0,
        )

    return pl.pallas_call(
        body,
        grid=(_NC, nb),
        in_specs=[
            pl.BlockSpec((bm, HID), lambda c, i: (c * nb + i, 0)),
            pl.BlockSpec((1, bm, HID), lambda c, i: (c, i, 0)),
            pl.BlockSpec((HID, HID), lambda c, i: (0, 0)),
            pl.BlockSpec((1, HID), lambda c, i: (0, 0)),
        ],
        out_specs=pl.BlockSpec((bm, HID), lambda c, i: (c * nb + i, 0)),
        out_shape=jax.ShapeDtypeStruct((N, HID), jnp.float32),
    )(h, acc2, Wu, bu)


def _dec(pool2, Wl, bl, W1, b1, W2, b2, W3, b3, Wn1, bn1, Wn2, bn2):
    def body(p_ref, wl_ref, bl_ref, w1_ref, b1_ref, w2_ref, b2_ref, w3_ref, b3_ref,
             wn1_ref, bn1_ref, wn2_ref, bn2_ref, z_ref, nf_ref, pn_ref):
        pooled = jnp.concatenate([p_ref[0, :_BHALF], p_ref[1, :_BHALF]], axis=0)
        z = jnp.dot(pooled, wl_ref[...], preferred_element_type=jnp.float32) + bl_ref[...]
        d1 = jnp.maximum(jnp.dot(z, w1_ref[...], preferred_element_type=jnp.float32) + b1_ref[...], 0.0)
        d2 = jnp.maximum(jnp.dot(d1, w2_ref[...], preferred_element_type=jnp.float32) + b2_ref[...], 0.0)
        nf = jnp.dot(d2, w3_ref[...], preferred_element_type=jnp.float32) + b3_ref[...]
        n1 = jnp.maximum(jnp.dot(z, wn1_ref[...], preferred_element_type=jnp.float32) + bn1_ref[...], 0.0)
        pn = jnp.maximum(jnp.dot(n1, wn2_ref[...], preferred_element_type=jnp.float32) + bn2_ref[...], 0.0)
        z_ref[...] = z
        nf_ref[...] = nf
        pn_ref[...] = pn

    return pl.pallas_call(
        body,
        out_shape=(
            jax.ShapeDtypeStruct((B, LAT), jnp.float32),
            jax.ShapeDtypeStruct((B, MAXN * F), jnp.float32),
            jax.ShapeDtypeStruct((B, 1), jnp.float32),
        ),
    )(pool2, Wl, bl, W1, b1, W2, b2, W3, b3, Wn1, bn1, Wn2, bn2)


def kernel(x, pos, batch, edge_index_3rd, W_emb, b_emb, W_filt, b_filt, W_upd, b_upd,
           W_lin1, b_lin1, Wd1, bd1, Wd2, bd2, Wd3, bd3, Wn1, bn1, Wn2, bn2):
    f32 = jnp.float32
    ei = edge_index_3rd.astype(jnp.int32)
    idx_all = ei.reshape(-1)
    dst = ei[0]
    src = ei[1]

    pos16 = jnp.pad(pos, ((0, 0), (0, 13)))
    pg = _sc_gather(pos16, idx_all, chunk=2000)
    geom = _geom(pg)

    h = _emb(x, W_emb, b_emb.reshape(1, HID))

    lo = jnp.where(dst < _RHALF, dst, _TRASH_E)
    hi = jnp.where(dst >= _RHALF, dst - _RHALF, _TRASH_E)
    idx2_e = jnp.stack([lo, hi]).astype(jnp.int32)
    zeros_e = jnp.zeros((_ZSL_E, HID), f32)

    offd = jnp.asarray(np.linspace(0.0, CUT, NG_D), f32).reshape(1, NG_D)
    offt = jnp.asarray(np.linspace(0.0, np.pi, NG_T), f32).reshape(1, NG_T)
    offp = jnp.asarray(np.linspace(-np.pi, np.pi, NG_P), f32).reshape(1, NG_P)

    for t in range(NI):
        hg = _sc_gather(h, src, chunk=1000)
        msg = _msg(geom, hg, W_filt[t, :NG_D], W_filt[t, NG_D:NG_D + NG_T],
                   W_filt[t, NG_D + NG_T:], b_filt[t].reshape(1, HID),
                   offd, offt, offp)
        acc2 = _sc_scatter(msg, idx2_e, zeros_e, racc=_RACC_E, chunk=400)
        h = _upd(h, acc2, W_upd[t], b_upd[t].reshape(1, HID))

    bat = batch.astype(jnp.int32)
    pad = _NPAD - N
    blo = jnp.concatenate([jnp.where(bat < _BHALF, bat, _TRASH_B),
                           jnp.full((pad,), _TRASH_B, jnp.int32)])
    bhi = jnp.concatenate([jnp.where(bat >= _BHALF, bat - _BHALF, _TRASH_B),
                           jnp.full((pad,), _TRASH_B, jnp.int32)])
    idx2_b = jnp.stack([blo, bhi]).astype(jnp.int32)
    hp = jnp.pad(h, ((0, pad), (0, 0)))
    pool2 = _sc_scatter(hp, idx2_b, jnp.zeros((_ZSL_B, HID), f32),
                        racc=_RACC_B, chunk=392)

    z, nf, pn = _dec(pool2, W_lin1, b_lin1.reshape(1, LAT),
                     Wd1, bd1.reshape(1, HID * 2), Wd2, bd2.reshape(1, HID * 4),
                     Wd3, bd3.reshape(1, MAXN * F), Wn1, bn1.reshape(1, HID),
                     Wn2, bn2.reshape(1, 1))
    return nf.reshape(B, MAXN, F), z, pn


# speedup vs baseline: 3.4632x; 1.6206x over previous
import functools

import numpy as np
import jax
import jax.numpy as jnp
from jax import lax
from jax.experimental import pallas as pl
from jax.experimental.pallas import tpu as pltpu
from jax.experimental.pallas import tpu_sc as plsc

N = 50000
E = 800000
B = 500
F = 5
HID = 64
LAT = 64
MAXN = 150
CUT = 10.0
NI = 3
NG_D, NG_T, NG_P = 50, 6, 12

_NC = 2
_NS = 16
_NW = _NC * _NS

_RHALF = N // _NC
_RACC_E = 25024
_TRASH_E = 25008
_ZSL_E = _RACC_E // _NS

_BHALF = B // _NC
_RACC_B = 256
_TRASH_B = 252
_ZSL_B = _RACC_B // _NS
_NPAD = 50176


def _sc_gather(table, idx, chunk):
    M = idx.shape[0]
    D = table.shape[1]
    m = M // _NW
    steps = m // chunk
    mesh = plsc.VectorSubcoreMesh(core_axis_name="c", subcore_axis_name="s")

    @functools.partial(
        pl.kernel,
        mesh=mesh,
        out_type=jax.ShapeDtypeStruct((M, D), jnp.float32),
        compiler_params=pltpu.CompilerParams(use_tc_tiling_on_sc=False),
        scratch_types=[
            pltpu.VMEM((chunk,), jnp.int32),
            pltpu.VMEM((chunk, D), jnp.float32),
            pltpu.SemaphoreType.DMA,
        ],
    )
    def k(table_h, idx_h, out_h, idx_v, rows_v, sem):
        wid = lax.axis_index("s") * _NC + lax.axis_index("c")
        base = wid * m

        def body(t, carry):
            off = base + t * chunk
            pltpu.sync_copy(idx_h.at[pl.ds(off, chunk)], idx_v)
            pltpu.async_copy(table_h.at[idx_v], rows_v, sem).wait()
            pltpu.sync_copy(rows_v, out_h.at[pl.ds(off, chunk)])
            return carry

        lax.fori_loop(0, steps, body, 0)

    return k(table, idx)


def _sc_scatter(rows, idx2, zeros, racc, chunk):
    M = rows.shape[0]
    m = M // _NS
    steps = m // chunk
    zsl = racc // _NS
    mesh = plsc.VectorSubcoreMesh(core_axis_name="c", subcore_axis_name="s")

    @functools.partial(
        pl.kernel,
        mesh=mesh,
        out_type=jax.ShapeDtypeStruct((_NC, racc, 64), jnp.float32),
        compiler_params=pltpu.CompilerParams(use_tc_tiling_on_sc=False),
        scratch_types=[
            pltpu.VMEM((chunk,), jnp.int32),
            pltpu.VMEM((chunk, 64), jnp.float32),
            pltpu.VMEM_SHARED((racc, 64), jnp.float32),
            pltpu.SemaphoreType.DMA,
        ],
    )
    def k(rows_h, idx2_h, z_h, out_h, idx_v, rows_v, acc_s, sem):
        c = lax.axis_index("c")
        s = lax.axis_index("s")
        pltpu.sync_copy(z_h, acc_s.at[pl.ds(s * zsl, zsl)])
        plsc.subcore_barrier()

        def body(t, carry):
            off = s * m + t * chunk
            pltpu.sync_copy(rows_h.at[pl.ds(off, chunk)], rows_v)
            pltpu.sync_copy(idx2_h.at[c, pl.ds(off, chunk)], idx_v)
            pltpu.sync_copy(rows_v, acc_s.at[idx_v], add=True)
            return carry

        lax.fori_loop(0, steps, body, 0)
        plsc.subcore_barrier()
        pltpu.sync_copy(acc_s.at[pl.ds(s * zsl, zsl)], out_h.at[c, pl.ds(s * zsl, zsl)])

    return k(rows, idx2, zeros)


def _emb(x, W, b2):
    bm = 5000

    def body(x_ref, w_ref, b_ref, o_ref):
        o_ref[...] = (
            jnp.dot(x_ref[...], w_ref[...], preferred_element_type=jnp.float32)
            + b_ref[...]
        )

    return pl.pallas_call(
        body,
        grid=(N // bm,),
        in_specs=[
            pl.BlockSpec((bm, F), lambda i: (i, 0)),
            pl.BlockSpec((F, HID), lambda i: (0, 0)),
            pl.BlockSpec((1, HID), lambda i: (0, 0)),
        ],
        out_specs=pl.BlockSpec((bm, HID), lambda i: (i, 0)),
        out_shape=jax.ShapeDtypeStruct((N, HID), jnp.float32),
    )(x, W, b2)


def _geom_filt(pg, Wt50s, Wt6s, Wt12s, bfs, offd_c, offt_c, offp_c):
    bE = 2000
    nb = E // bE

    def body(pi_ref, pj_ref, pk_ref, pl_ref, w50_ref, w6_ref, w12_ref, b_ref,
             od_ref, ot_ref, op_ref, o0_ref, o1_ref, o2_ref):
        eps = 1e-8
        b1 = jnp.transpose(pj_ref[...] - pi_ref[...])[:3]
        b2 = jnp.transpose(pk_ref[...] - pj_ref[...])[:3]
        b3 = jnp.transpose(pl_ref[...] - pk_ref[...])[:3]
        x1, y1, z1 = b1[0:1], b1[1:2], b1[2:3]
        x2, y2, z2 = b2[0:1], b2[1:2], b2[2:3]
        x3, y3, z3 = b3[0:1], b3[1:2], b3[2:3]
        s11 = x1 * x1 + y1 * y1 + z1 * z1
        s12 = x1 * x2 + y1 * y2 + z1 * z2
        s22 = x2 * x2 + y2 * y2 + z2 * z2
        s23 = x2 * x3 + y2 * y3 + z2 * z3
        s13 = x1 * x3 + y1 * y3 + z1 * z3
        d3 = (x1 * (y2 * z3 - z2 * y3)
              + y1 * (z2 * x3 - x2 * z3)
              + z1 * (x2 * y3 - y2 * x3))
        dist = jnp.sqrt(s11 + eps)
        nu = jnp.sqrt(s11)
        nv = jnp.sqrt(s22)
        cos_t = -s12 / (nu * nv + eps)
        cos_t = jnp.clip(cos_t, -1.0 + 1e-7, 1.0 - 1e-7)
        theta = jnp.arctan2(jnp.sqrt(1.0 - cos_t * cos_t), cos_t)
        tx = (s12 * s23 - s13 * s22) + eps
        ty = -(d3 * s22) / (nv + eps)
        phi = jnp.arctan2(ty, tx)
        C = 0.5 * (jnp.cos(dist * (np.pi / CUT)) + 1.0) * (dist < CUT).astype(jnp.float32)
        rbf = jnp.exp(_CD * (dist - od_ref[...]) ** 2)
        tbf = jnp.exp(_CT * (theta - ot_ref[...]) ** 2)
        pbf = jnp.exp(_CP * (phi - op_ref[...]) ** 2)
        outs = (o0_ref, o1_ref, o2_ref)
        for t in range(NI):
            acc = (
                jnp.dot(w50_ref[t], rbf, preferred_element_type=jnp.float32)
                + jnp.dot(w6_ref[t], tbf, preferred_element_type=jnp.float32)
                + jnp.dot(w12_ref[t], pbf, preferred_element_type=jnp.float32)
                + b_ref[t]
            )
            outs[t][...] = jnp.transpose(jnp.maximum(acc, 0.0) * C)

    out = pl.pallas_call(
        body,
        grid=(nb,),
        in_specs=[
            pl.BlockSpec((bE, 16), lambda i, q=q: (i + q * nb, 0)) for q in range(4)
        ] + [
            pl.BlockSpec((NI, HID, NG_D), lambda i: (0, 0, 0)),
            pl.BlockSpec((NI, HID, NG_T), lambda i: (0, 0, 0)),
            pl.BlockSpec((NI, HID, NG_P), lambda i: (0, 0, 0)),
            pl.BlockSpec((NI, HID, 1), lambda i: (0, 0, 0)),
            pl.BlockSpec((NG_D, 1), lambda i: (0, 0)),
            pl.BlockSpec((NG_T, 1), lambda i: (0, 0)),
            pl.BlockSpec((NG_P, 1), lambda i: (0, 0)),
        ],
        out_specs=[pl.BlockSpec((bE, HID), lambda i: (i, 0)) for _ in range(NI)],
        out_shape=[jax.ShapeDtypeStruct((E, HID), jnp.float32) for _ in range(NI)],
    )(pg, pg, pg, pg, Wt50s, Wt6s, Wt12s, bfs, offd_c, offt_c, offp_c)
    return out


def _mul(hg, filt):
    bE = 8000
    nb = E // bE

    def body(a_ref, b_ref, o_ref):
        o_ref[...] = a_ref[...] * b_ref[...]

    return pl.pallas_call(
        body,
        grid=(nb,),
        in_specs=[
            pl.BlockSpec((bE, HID), lambda i: (i, 0)),
            pl.BlockSpec((bE, HID), lambda i: (i, 0)),
        ],
        out_specs=pl.BlockSpec((bE, HID), lambda i: (i, 0)),
        out_shape=jax.ShapeDtypeStruct((E, HID), jnp.float32),
    )(hg, filt)


_CD = float(-0.5 / (CUT / (NG_D - 1)) ** 2)
_CT = float(-0.5 / (np.pi / (NG_T - 1)) ** 2)
_CP = float(-0.5 / (2.0 * np.pi / (NG_P - 1)) ** 2)


def _upd(h, acc2, Wu, bu):
    bm = 5000
    nb = _RHALF // bm

    def body(h_ref, a_ref, w_ref, b_ref, o_ref):
        a = a_ref[0]
        o_ref[...] = h_ref[...] + jnp.maximum(
            jnp.dot(a, w_ref[...], preferred_element_type=jnp.float32) + b_ref[...],
            0.0,
        )

    return pl.pallas_call(
        body,
        grid=(_NC, nb),
        in_specs=[
            pl.BlockSpec((bm, HID), lambda c, i: (c * nb + i, 0)),
            pl.BlockSpec((1, bm, HID), lambda c, i: (c, i, 0)),
            pl.BlockSpec((HID, HID), lambda c, i: (0, 0)),
            pl.BlockSpec((1, HID), lambda c, i: (0, 0)),
        ],
        out_specs=pl.BlockSpec((bm, HID), lambda c, i: (c * nb + i, 0)),
        out_shape=jax.ShapeDtypeStruct((N, HID), jnp.float32),
    )(h, acc2, Wu, bu)


def _dec(pool2, Wl, bl, W1, b1, W2, b2, W3, b3, Wn1, bn1, Wn2, bn2):
    def body(p_ref, wl_ref, bl_ref, w1_ref, b1_ref, w2_ref, b2_ref, w3_ref, b3_ref,
             wn1_ref, bn1_ref, wn2_ref, bn2_ref, z_ref, nf_ref, pn_ref):
        pooled = jnp.concatenate([p_ref[0, :_BHALF], p_ref[1, :_BHALF]], axis=0)
        z = jnp.dot(pooled, wl_ref[...], preferred_element_type=jnp.float32) + bl_ref[...]
        d1 = jnp.maximum(jnp.dot(z, w1_ref[...], preferred_element_type=jnp.float32) + b1_ref[...], 0.0)
        d2 = jnp.maximum(jnp.dot(d1, w2_ref[...], preferred_element_type=jnp.float32) + b2_ref[...], 0.0)
        nf = jnp.dot(d2, w3_ref[...], preferred_element_type=jnp.float32) + b3_ref[...]
        n1 = jnp.maximum(jnp.dot(z, wn1_ref[...], preferred_element_type=jnp.float32) + bn1_ref[...], 0.0)
        pn = jnp.maximum(jnp.dot(n1, wn2_ref[...], preferred_element_type=jnp.float32) + bn2_ref[...], 0.0)
        z_ref[...] = z
        nf_ref[...] = nf
        pn_ref[...] = pn

    return pl.pallas_call(
        body,
        out_shape=(
            jax.ShapeDtypeStruct((B, LAT), jnp.float32),
            jax.ShapeDtypeStruct((B, MAXN * F), jnp.float32),
            jax.ShapeDtypeStruct((B, 1), jnp.float32),
        ),
    )(pool2, Wl, bl, W1, b1, W2, b2, W3, b3, Wn1, bn1, Wn2, bn2)


def kernel(x, pos, batch, edge_index_3rd, W_emb, b_emb, W_filt, b_filt, W_upd, b_upd,
           W_lin1, b_lin1, Wd1, bd1, Wd2, bd2, Wd3, bd3, Wn1, bn1, Wn2, bn2):
    f32 = jnp.float32
    ei = edge_index_3rd.astype(jnp.int32)
    idx_all = ei.reshape(-1)
    dst = ei[0]
    src = ei[1]

    pos16 = jnp.pad(pos, ((0, 0), (0, 13)))
    pg = _sc_gather(pos16, idx_all, chunk=2000)

    offd_c = jnp.asarray(np.linspace(0.0, CUT, NG_D), f32).reshape(NG_D, 1)
    offt_c = jnp.asarray(np.linspace(0.0, np.pi, NG_T), f32).reshape(NG_T, 1)
    offp_c = jnp.asarray(np.linspace(-np.pi, np.pi, NG_P), f32).reshape(NG_P, 1)
    Wt50s = jnp.transpose(W_filt[:, :NG_D, :], (0, 2, 1))
    Wt6s = jnp.transpose(W_filt[:, NG_D:NG_D + NG_T, :], (0, 2, 1))
    Wt12s = jnp.transpose(W_filt[:, NG_D + NG_T:, :], (0, 2, 1))
    bfs = b_filt.reshape(NI, HID, 1)
    filts = _geom_filt(pg, Wt50s, Wt6s, Wt12s, bfs, offd_c, offt_c, offp_c)

    h = _emb(x, W_emb, b_emb.reshape(1, HID))

    lo = jnp.where(dst < _RHALF, dst, _TRASH_E)
    hi = jnp.where(dst >= _RHALF, dst - _RHALF, _TRASH_E)
    idx2_e = jnp.stack([lo, hi]).astype(jnp.int32)
    zeros_e = jnp.zeros((_ZSL_E, HID), f32)

    for t in range(NI):
        hg = _sc_gather(h, src, chunk=1000)
        msg = _mul(hg, filts[t])
        acc2 = _sc_scatter(msg, idx2_e, zeros_e, racc=_RACC_E, chunk=400)
        h = _upd(h, acc2, W_upd[t], b_upd[t].reshape(1, HID))

    bat = batch.astype(jnp.int32)
    pad = _NPAD - N
    blo = jnp.concatenate([jnp.where(bat < _BHALF, bat, _TRASH_B),
                           jnp.full((pad,), _TRASH_B, jnp.int32)])
    bhi = jnp.concatenate([jnp.where(bat >= _BHALF, bat - _BHALF, _TRASH_B),
                           jnp.full((pad,), _TRASH_B, jnp.int32)])
    idx2_b = jnp.stack([blo, bhi]).astype(jnp.int32)
    hp = jnp.pad(h, ((0, pad), (0, 0)))
    pool2 = _sc_scatter(hp, idx2_b, jnp.zeros((_ZSL_B, HID), f32),
                        racc=_RACC_B, chunk=392)

    z, nf, pn = _dec(pool2, W_lin1, b_lin1.reshape(1, LAT),
                     Wd1, bd1.reshape(1, HID * 2), Wd2, bd2.reshape(1, HID * 4),
                     Wd3, bd3.reshape(1, MAXN * F), Wn1, bn1.reshape(1, HID),
                     Wn2, bn2.reshape(1, 1))
    return nf.reshape(B, MAXN, F), z, pn


# speedup vs baseline: 3.5020x; 1.0112x over previous
import functools

import numpy as np
import jax
import jax.numpy as jnp
from jax import lax
from jax.experimental import pallas as pl
from jax.experimental.pallas import tpu as pltpu
from jax.experimental.pallas import tpu_sc as plsc

N = 50000
E = 800000
B = 500
F = 5
HID = 64
LAT = 64
MAXN = 150
CUT = 10.0
NI = 3
NG_D, NG_T, NG_P = 50, 6, 12

_NC = 2
_NS = 16
_NW = _NC * _NS

_RHALF = N // _NC
_RACC_E = 25024
_TRASH_E = 25008
_ZSL_E = _RACC_E // _NS

_BHALF = B // _NC
_RACC_B = 256
_TRASH_B = 252
_ZSL_B = _RACC_B // _NS
_NPAD = 50176


def _sc_gather(table, idx, chunk):
    M = idx.shape[0]
    D = table.shape[1]
    m = M // _NW
    steps = m // chunk
    mesh = plsc.VectorSubcoreMesh(core_axis_name="c", subcore_axis_name="s")

    @functools.partial(
        pl.kernel,
        mesh=mesh,
        out_type=jax.ShapeDtypeStruct((M, D), jnp.float32),
        compiler_params=pltpu.CompilerParams(use_tc_tiling_on_sc=False),
        scratch_types=[
            pltpu.VMEM((chunk,), jnp.int32),
            pltpu.VMEM((chunk, D), jnp.float32),
            pltpu.SemaphoreType.DMA,
        ],
    )
    def k(table_h, idx_h, out_h, idx_v, rows_v, sem):
        wid = lax.axis_index("s") * _NC + lax.axis_index("c")
        base = wid * m

        def body(t, carry):
            off = base + t * chunk
            pltpu.sync_copy(idx_h.at[pl.ds(off, chunk)], idx_v)
            pltpu.async_copy(table_h.at[idx_v], rows_v, sem).wait()
            pltpu.sync_copy(rows_v, out_h.at[pl.ds(off, chunk)])
            return carry

        lax.fori_loop(0, steps, body, 0)

    return k(table, idx)


def _sc_scatter(rows, idx2, zeros, racc, chunk):
    M = rows.shape[0]
    m = M // _NS
    steps = m // chunk
    zsl = racc // _NS
    mesh = plsc.VectorSubcoreMesh(core_axis_name="c", subcore_axis_name="s")

    @functools.partial(
        pl.kernel,
        mesh=mesh,
        out_type=jax.ShapeDtypeStruct((_NC, racc, 64), jnp.float32),
        compiler_params=pltpu.CompilerParams(use_tc_tiling_on_sc=False),
        scratch_types=[
            pltpu.VMEM((2, chunk), jnp.int32),
            pltpu.VMEM((2, chunk, 64), jnp.float32),
            pltpu.VMEM_SHARED((racc, 64), jnp.float32),
            pltpu.SemaphoreType.DMA((2,)),
            pltpu.SemaphoreType.DMA((2,)),
        ],
    )
    def k(rows_h, idx2_h, z_h, out_h, idx_v, rows_v, acc_s, rsem, isem):
        c = lax.axis_index("c")
        s = lax.axis_index("s")
        pltpu.sync_copy(z_h, acc_s.at[pl.ds(s * zsl, zsl)])
        plsc.subcore_barrier()

        def issue(t):
            slot = lax.rem(t, 2)
            off = s * m + t * chunk
            pltpu.make_async_copy(
                rows_h.at[pl.ds(off, chunk)], rows_v.at[slot], rsem.at[slot]
            ).start()
            pltpu.make_async_copy(
                idx2_h.at[c, pl.ds(off, chunk)], idx_v.at[slot], isem.at[slot]
            ).start()

        issue(0)

        def body(t, carry):
            slot = lax.rem(t, 2)

            @pl.when(t + 1 < steps)
            def _():
                issue(t + 1)

            off = s * m + t * chunk
            pltpu.make_async_copy(
                rows_h.at[pl.ds(off, chunk)], rows_v.at[slot], rsem.at[slot]
            ).wait()
            pltpu.make_async_copy(
                idx2_h.at[c, pl.ds(off, chunk)], idx_v.at[slot], isem.at[slot]
            ).wait()
            pltpu.sync_copy(rows_v.at[slot], acc_s.at[idx_v.at[slot]], add=True)
            return carry

        lax.fori_loop(0, steps, body, 0)
        plsc.subcore_barrier()
        pltpu.sync_copy(acc_s.at[pl.ds(s * zsl, zsl)], out_h.at[c, pl.ds(s * zsl, zsl)])

    return k(rows, idx2, zeros)


def _emb(x, W, b2):
    bm = 5000

    def body(x_ref, w_ref, b_ref, o_ref):
        o_ref[...] = (
            jnp.dot(x_ref[...], w_ref[...], preferred_element_type=jnp.float32)
            + b_ref[...]
        )

    return pl.pallas_call(
        body,
        grid=(N // bm,),
        in_specs=[
            pl.BlockSpec((bm, F), lambda i: (i, 0)),
            pl.BlockSpec((F, HID), lambda i: (0, 0)),
            pl.BlockSpec((1, HID), lambda i: (0, 0)),
        ],
        out_specs=pl.BlockSpec((bm, HID), lambda i: (i, 0)),
        out_shape=jax.ShapeDtypeStruct((N, HID), jnp.float32),
    )(x, W, b2)


def _geom_filt(pg, Wt50s, Wt6s, Wt12s, bfs, offd_c, offt_c, offp_c):
    bE = 2000
    nb = E // bE

    def body(pi_ref, pj_ref, pk_ref, pl_ref, w50_ref, w6_ref, w12_ref, b_ref,
             od_ref, ot_ref, op_ref, o0_ref, o1_ref, o2_ref):
        eps = 1e-8
        b1 = jnp.transpose(pj_ref[...] - pi_ref[...])[:3]
        b2 = jnp.transpose(pk_ref[...] - pj_ref[...])[:3]
        b3 = jnp.transpose(pl_ref[...] - pk_ref[...])[:3]
        x1, y1, z1 = b1[0:1], b1[1:2], b1[2:3]
        x2, y2, z2 = b2[0:1], b2[1:2], b2[2:3]
        x3, y3, z3 = b3[0:1], b3[1:2], b3[2:3]
        s11 = x1 * x1 + y1 * y1 + z1 * z1
        s12 = x1 * x2 + y1 * y2 + z1 * z2
        s22 = x2 * x2 + y2 * y2 + z2 * z2
        s23 = x2 * x3 + y2 * y3 + z2 * z3
        s13 = x1 * x3 + y1 * y3 + z1 * z3
        d3 = (x1 * (y2 * z3 - z2 * y3)
              + y1 * (z2 * x3 - x2 * z3)
              + z1 * (x2 * y3 - y2 * x3))
        dist = jnp.sqrt(s11 + eps)
        nu = jnp.sqrt(s11)
        nv = jnp.sqrt(s22)
        cos_t = -s12 / (nu * nv + eps)
        cos_t = jnp.clip(cos_t, -1.0 + 1e-7, 1.0 - 1e-7)
        theta = jnp.arctan2(jnp.sqrt(1.0 - cos_t * cos_t), cos_t)
        tx = (s12 * s23 - s13 * s22) + eps
        ty = -(d3 * s22) / (nv + eps)
        phi = jnp.arctan2(ty, tx)
        C = 0.5 * (jnp.cos(dist * (np.pi / CUT)) + 1.0) * (dist < CUT).astype(jnp.float32)
        rbf = jnp.exp(_CD * (dist - od_ref[...]) ** 2)
        tbf = jnp.exp(_CT * (theta - ot_ref[...]) ** 2)
        pbf = jnp.exp(_CP * (phi - op_ref[...]) ** 2)
        outs = (o0_ref, o1_ref, o2_ref)
        for t in range(NI):
            acc = (
                jnp.dot(w50_ref[t], rbf, preferred_element_type=jnp.float32)
                + jnp.dot(w6_ref[t], tbf, preferred_element_type=jnp.float32)
                + jnp.dot(w12_ref[t], pbf, preferred_element_type=jnp.float32)
                + b_ref[t]
            )
            outs[t][...] = jnp.transpose(jnp.maximum(acc, 0.0) * C)

    out = pl.pallas_call(
        body,
        grid=(nb,),
        in_specs=[
            pl.BlockSpec((bE, 16), lambda i, q=q: (i + q * nb, 0)) for q in range(4)
        ] + [
            pl.BlockSpec((NI, HID, NG_D), lambda i: (0, 0, 0)),
            pl.BlockSpec((NI, HID, NG_T), lambda i: (0, 0, 0)),
            pl.BlockSpec((NI, HID, NG_P), lambda i: (0, 0, 0)),
            pl.BlockSpec((NI, HID, 1), lambda i: (0, 0, 0)),
            pl.BlockSpec((NG_D, 1), lambda i: (0, 0)),
            pl.BlockSpec((NG_T, 1), lambda i: (0, 0)),
            pl.BlockSpec((NG_P, 1), lambda i: (0, 0)),
        ],
        out_specs=[pl.BlockSpec((bE, HID), lambda i: (i, 0)) for _ in range(NI)],
        out_shape=[jax.ShapeDtypeStruct((E, HID), jnp.float32) for _ in range(NI)],
    )(pg, pg, pg, pg, Wt50s, Wt6s, Wt12s, bfs, offd_c, offt_c, offp_c)
    return out


def _mul(hg, filt):
    bE = 8000
    nb = E // bE

    def body(a_ref, b_ref, o_ref):
        o_ref[...] = a_ref[...] * b_ref[...]

    return pl.pallas_call(
        body,
        grid=(nb,),
        in_specs=[
            pl.BlockSpec((bE, HID), lambda i: (i, 0)),
            pl.BlockSpec((bE, HID), lambda i: (i, 0)),
        ],
        out_specs=pl.BlockSpec((bE, HID), lambda i: (i, 0)),
        out_shape=jax.ShapeDtypeStruct((E, HID), jnp.float32),
    )(hg, filt)


_CD = float(-0.5 / (CUT / (NG_D - 1)) ** 2)
_CT = float(-0.5 / (np.pi / (NG_T - 1)) ** 2)
_CP = float(-0.5 / (2.0 * np.pi / (NG_P - 1)) ** 2)


def _upd(h, acc2, Wu, bu):
    bm = 5000
    nb = _RHALF // bm

    def body(h_ref, a_ref, w_ref, b_ref, o_ref):
        a = a_ref[0]
        o_ref[...] = h_ref[...] + jnp.maximum(
            jnp.dot(a, w_ref[...], preferred_element_type=jnp.float32) + b_ref[...],
            0.0,
        )

    return pl.pallas_call(
        body,
        grid=(_NC, nb),
        in_specs=[
            pl.BlockSpec((bm, HID), lambda c, i: (c * nb + i, 0)),
            pl.BlockSpec((1, bm, HID), lambda c, i: (c, i, 0)),
            pl.BlockSpec((HID, HID), lambda c, i: (0, 0)),
            pl.BlockSpec((1, HID), lambda c, i: (0, 0)),
        ],
        out_specs=pl.BlockSpec((bm, HID), lambda c, i: (c * nb + i, 0)),
        out_shape=jax.ShapeDtypeStruct((N, HID), jnp.float32),
    )(h, acc2, Wu, bu)


def _dec(pool2, Wl, bl, W1, b1, W2, b2, W3, b3, Wn1, bn1, Wn2, bn2):
    def body(p_ref, wl_ref, bl_ref, w1_ref, b1_ref, w2_ref, b2_ref, w3_ref, b3_ref,
             wn1_ref, bn1_ref, wn2_ref, bn2_ref, z_ref, nf_ref, pn_ref):
        pooled = jnp.concatenate([p_ref[0, :_BHALF], p_ref[1, :_BHALF]], axis=0)
        z = jnp.dot(pooled, wl_ref[...], preferred_element_type=jnp.float32) + bl_ref[...]
        d1 = jnp.maximum(jnp.dot(z, w1_ref[...], preferred_element_type=jnp.float32) + b1_ref[...], 0.0)
        d2 = jnp.maximum(jnp.dot(d1, w2_ref[...], preferred_element_type=jnp.float32) + b2_ref[...], 0.0)
        nf = jnp.dot(d2, w3_ref[...], preferred_element_type=jnp.float32) + b3_ref[...]
        n1 = jnp.maximum(jnp.dot(z, wn1_ref[...], preferred_element_type=jnp.float32) + bn1_ref[...], 0.0)
        pn = jnp.maximum(jnp.dot(n1, wn2_ref[...], preferred_element_type=jnp.float32) + bn2_ref[...], 0.0)
        z_ref[...] = z
        nf_ref[...] = nf
        pn_ref[...] = pn

    return pl.pallas_call(
        body,
        out_shape=(
            jax.ShapeDtypeStruct((B, LAT), jnp.float32),
            jax.ShapeDtypeStruct((B, MAXN * F), jnp.float32),
            jax.ShapeDtypeStruct((B, 1), jnp.float32),
        ),
    )(pool2, Wl, bl, W1, b1, W2, b2, W3, b3, Wn1, bn1, Wn2, bn2)


def kernel(x, pos, batch, edge_index_3rd, W_emb, b_emb, W_filt, b_filt, W_upd, b_upd,
           W_lin1, b_lin1, Wd1, bd1, Wd2, bd2, Wd3, bd3, Wn1, bn1, Wn2, bn2):
    f32 = jnp.float32
    ei = edge_index_3rd.astype(jnp.int32)
    idx_all = ei.reshape(-1)
    dst = ei[0]
    src = ei[1]

    pos16 = jnp.pad(pos, ((0, 0), (0, 13)))
    pg = _sc_gather(pos16, idx_all, chunk=2000)

    offd_c = jnp.asarray(np.linspace(0.0, CUT, NG_D), f32).reshape(NG_D, 1)
    offt_c = jnp.asarray(np.linspace(0.0, np.pi, NG_T), f32).reshape(NG_T, 1)
    offp_c = jnp.asarray(np.linspace(-np.pi, np.pi, NG_P), f32).reshape(NG_P, 1)
    Wt50s = jnp.transpose(W_filt[:, :NG_D, :], (0, 2, 1))
    Wt6s = jnp.transpose(W_filt[:, NG_D:NG_D + NG_T, :], (0, 2, 1))
    Wt12s = jnp.transpose(W_filt[:, NG_D + NG_T:, :], (0, 2, 1))
    bfs = b_filt.reshape(NI, HID, 1)
    filts = _geom_filt(pg, Wt50s, Wt6s, Wt12s, bfs, offd_c, offt_c, offp_c)

    h = _emb(x, W_emb, b_emb.reshape(1, HID))

    lo = jnp.where(dst < _RHALF, dst, _TRASH_E)
    hi = jnp.where(dst >= _RHALF, dst - _RHALF, _TRASH_E)
    idx2_e = jnp.stack([lo, hi]).astype(jnp.int32)
    zeros_e = jnp.zeros((_ZSL_E, HID), f32)

    for t in range(NI):
        hg = _sc_gather(h, src, chunk=1000)
        msg = _mul(hg, filts[t])
        acc2 = _sc_scatter(msg, idx2_e, zeros_e, racc=_RACC_E, chunk=200)
        h = _upd(h, acc2, W_upd[t], b_upd[t].reshape(1, HID))

    bat = batch.astype(jnp.int32)
    pad = _NPAD - N
    blo = jnp.concatenate([jnp.where(bat < _BHALF, bat, _TRASH_B),
                           jnp.full((pad,), _TRASH_B, jnp.int32)])
    bhi = jnp.concatenate([jnp.where(bat >= _BHALF, bat - _BHALF, _TRASH_B),
                           jnp.full((pad,), _TRASH_B, jnp.int32)])
    idx2_b = jnp.stack([blo, bhi]).astype(jnp.int32)
    hp = jnp.pad(h, ((0, pad), (0, 0)))
    pool2 = _sc_scatter(hp, idx2_b, jnp.zeros((_ZSL_B, HID), f32),
                        racc=_RACC_B, chunk=392)

    z, nf, pn = _dec(pool2, W_lin1, b_lin1.reshape(1, LAT),
                     Wd1, bd1.reshape(1, HID * 2), Wd2, bd2.reshape(1, HID * 4),
                     Wd3, bd3.reshape(1, MAXN * F), Wn1, bn1.reshape(1, HID),
                     Wn2, bn2.reshape(1, 1))
    return nf.reshape(B, MAXN, F), z, pn


# speedup vs baseline: 4.1976x; 1.1986x over previous
import functools

import numpy as np
import jax
import jax.numpy as jnp
from jax import lax
from jax.experimental import pallas as pl
from jax.experimental.pallas import tpu as pltpu
from jax.experimental.pallas import tpu_sc as plsc

N = 50000
E = 800000
B = 500
F = 5
HID = 64
LAT = 64
MAXN = 150
CUT = 10.0
NI = 3
NG_D, NG_T, NG_P = 50, 6, 12

_NC = 2
_NS = 16
_NW = _NC * _NS

_RHALF = N // _NC
_RACC_E = 25024
_TRASH_E = 25008
_ZSL_E = _RACC_E // _NS

_BHALF = B // _NC
_RACC_B = 256
_TRASH_B = 252
_ZSL_B = _RACC_B // _NS
_NPAD = 50176


def _sc_gather(table, idx, chunk):
    M = idx.shape[0]
    D = table.shape[1]
    m = M // _NW
    steps = m // chunk
    mesh = plsc.VectorSubcoreMesh(core_axis_name="c", subcore_axis_name="s")

    @functools.partial(
        pl.kernel,
        mesh=mesh,
        out_type=jax.ShapeDtypeStruct((M, D), jnp.float32),
        compiler_params=pltpu.CompilerParams(use_tc_tiling_on_sc=False),
        scratch_types=[
            pltpu.VMEM((chunk,), jnp.int32),
            pltpu.VMEM((chunk, D), jnp.float32),
            pltpu.SemaphoreType.DMA,
        ],
    )
    def k(table_h, idx_h, out_h, idx_v, rows_v, sem):
        wid = lax.axis_index("s") * _NC + lax.axis_index("c")
        base = wid * m

        def body(t, carry):
            off = base + t * chunk
            pltpu.sync_copy(idx_h.at[pl.ds(off, chunk)], idx_v)
            pltpu.async_copy(table_h.at[idx_v], rows_v, sem).wait()
            pltpu.sync_copy(rows_v, out_h.at[pl.ds(off, chunk)])
            return carry

        lax.fori_loop(0, steps, body, 0)

    return k(table, idx)


_GEO_CHUNK = 1000
_GEO_PAD = 1008


def _sc_geo(pos16, ei):
    chunk = _GEO_CHUNK
    m = E // _NW
    steps = m // chunk
    ngr = _GEO_PAD // 16
    mesh = plsc.VectorSubcoreMesh(core_axis_name="c", subcore_axis_name="s")

    @functools.partial(
        pl.kernel,
        mesh=mesh,
        out_type=jax.ShapeDtypeStruct((E // chunk, 8, _GEO_PAD), jnp.float32),
        compiler_params=pltpu.CompilerParams(
            use_tc_tiling_on_sc=False, needs_layout_passes=False
        ),
        scratch_types=[
            pltpu.VMEM((4, chunk), jnp.int32),
            pltpu.VMEM((4, chunk, 16), jnp.float32),
            pltpu.VMEM((8, _GEO_PAD), jnp.float32),
            pltpu.SemaphoreType.DMA((4,)),
        ],
    )
    def k(pos_h, ei_h, out_h, idx_v, pts_v, out_v, sems):
        wid = lax.axis_index("s") * _NC + lax.axis_index("c")

        def body(t, carry):
            off = wid * m + t * chunk
            for q in range(4):
                pltpu.sync_copy(ei_h.at[q, pl.ds(off, chunk)], idx_v.at[q])
            for q in range(4):
                pltpu.make_async_copy(
                    pos_h.at[idx_v.at[q]], pts_v.at[q], sems.at[q]
                ).start()
            for q in range(4):
                pltpu.make_async_copy(
                    pos_h.at[idx_v.at[q]], pts_v.at[q], sems.at[q]
                ).wait()

            def group(g, carry2):
                e0 = g * 16
                lane = lax.iota(jnp.int32, 16)
                ids = jnp.minimum(e0 + lane, chunk - 1)
                coords = []
                for q in range(4):
                    qv = jnp.full((16,), q, jnp.int32)
                    coords.append([
                        plsc.load_gather(
                            pts_v, [qv, ids, jnp.full((16,), cc, jnp.int32)]
                        )
                        for cc in range(3)
                    ])
                b1 = [coords[1][d] - coords[0][d] for d in range(3)]
                b2 = [coords[2][d] - coords[1][d] for d in range(3)]
                b3 = [coords[3][d] - coords[2][d] for d in range(3)]

                def dot(a, b):
                    return a[0] * b[0] + a[1] * b[1] + a[2] * b[2]

                d3 = (b1[0] * (b2[1] * b3[2] - b2[2] * b3[1])
                      + b1[1] * (b2[2] * b3[0] - b2[0] * b3[2])
                      + b1[2] * (b2[0] * b3[1] - b2[1] * b3[0]))
                vals = [dot(b1, b1), dot(b1, b2), dot(b2, b2),
                        dot(b2, b3), dot(b1, b3), d3]
                for r, v in enumerate(vals):
                    out_v[r, pl.ds(e0, 16)] = v
                return carry2

            lax.fori_loop(0, ngr, group, 0)
            pltpu.sync_copy(out_v, out_h.at[wid * steps + t])
            return carry

        lax.fori_loop(0, steps, body, 0)

    return k(pos16, ei)


def _sc_scatter(rows, idx2, zeros, racc, chunk):
    M = rows.shape[0]
    m = M // _NS
    steps = m // chunk
    zsl = racc // _NS
    mesh = plsc.VectorSubcoreMesh(core_axis_name="c", subcore_axis_name="s")

    @functools.partial(
        pl.kernel,
        mesh=mesh,
        out_type=jax.ShapeDtypeStruct((_NC, racc, 64), jnp.float32),
        compiler_params=pltpu.CompilerParams(use_tc_tiling_on_sc=False),
        scratch_types=[
            pltpu.VMEM((2, chunk), jnp.int32),
            pltpu.VMEM((2, chunk, 64), jnp.float32),
            pltpu.VMEM_SHARED((racc, 64), jnp.float32),
            pltpu.SemaphoreType.DMA((2,)),
            pltpu.SemaphoreType.DMA((2,)),
        ],
    )
    def k(rows_h, idx2_h, z_h, out_h, idx_v, rows_v, acc_s, rsem, isem):
        c = lax.axis_index("c")
        s = lax.axis_index("s")
        pltpu.sync_copy(z_h, acc_s.at[pl.ds(s * zsl, zsl)])
        plsc.subcore_barrier()

        def issue(t):
            slot = lax.rem(t, 2)
            off = s * m + t * chunk
            pltpu.make_async_copy(
                rows_h.at[pl.ds(off, chunk)], rows_v.at[slot], rsem.at[slot]
            ).start()
            pltpu.make_async_copy(
                idx2_h.at[c, pl.ds(off, chunk)], idx_v.at[slot], isem.at[slot]
            ).start()

        issue(0)

        def body(t, carry):
            slot = lax.rem(t, 2)

            @pl.when(t + 1 < steps)
            def _():
                issue(t + 1)

            off = s * m + t * chunk
            pltpu.make_async_copy(
                rows_h.at[pl.ds(off, chunk)], rows_v.at[slot], rsem.at[slot]
            ).wait()
            pltpu.make_async_copy(
                idx2_h.at[c, pl.ds(off, chunk)], idx_v.at[slot], isem.at[slot]
            ).wait()
            pltpu.sync_copy(rows_v.at[slot], acc_s.at[idx_v.at[slot]], add=True)
            return carry

        lax.fori_loop(0, steps, body, 0)
        plsc.subcore_barrier()
        pltpu.sync_copy(acc_s.at[pl.ds(s * zsl, zsl)], out_h.at[c, pl.ds(s * zsl, zsl)])

    return k(rows, idx2, zeros)


def _emb(x, W, b2):
    bm = 5000

    def body(x_ref, w_ref, b_ref, o_ref):
        o_ref[...] = (
            jnp.dot(x_ref[...], w_ref[...], preferred_element_type=jnp.float32)
            + b_ref[...]
        )

    return pl.pallas_call(
        body,
        grid=(N // bm,),
        in_specs=[
            pl.BlockSpec((bm, F), lambda i: (i, 0)),
            pl.BlockSpec((F, HID), lambda i: (0, 0)),
            pl.BlockSpec((1, HID), lambda i: (0, 0)),
        ],
        out_specs=pl.BlockSpec((bm, HID), lambda i: (i, 0)),
        out_shape=jax.ShapeDtypeStruct((N, HID), jnp.float32),
    )(x, W, b2)


def _geom_filt(geo, Wt50s, Wt6s, Wt12s, bfs, offd_c, offt_c, offp_c):
    bE = _GEO_CHUNK
    nb = E // bE

    def body(g_ref, w50_ref, w6_ref, w12_ref, b_ref,
             od_ref, ot_ref, op_ref, o0_ref, o1_ref, o2_ref):
        eps = 1e-8
        g = g_ref[0]
        s11 = g[0:1, :bE]
        s12 = g[1:2, :bE]
        s22 = g[2:3, :bE]
        s23 = g[3:4, :bE]
        s13 = g[4:5, :bE]
        d3 = g[5:6, :bE]
        dist = jnp.sqrt(s11 + eps)
        nu = jnp.sqrt(s11)
        nv = jnp.sqrt(s22)
        cos_t = -s12 / (nu * nv + eps)
        cos_t = jnp.clip(cos_t, -1.0 + 1e-7, 1.0 - 1e-7)
        theta = jnp.arctan2(jnp.sqrt(1.0 - cos_t * cos_t), cos_t)
        tx = (s12 * s23 - s13 * s22) + eps
        ty = -(d3 * s22) / (nv + eps)
        phi = jnp.arctan2(ty, tx)
        C = 0.5 * (jnp.cos(dist * (np.pi / CUT)) + 1.0) * (dist < CUT).astype(jnp.float32)
        rbf = jnp.exp(_CD * (dist - od_ref[...]) ** 2)
        tbf = jnp.exp(_CT * (theta - ot_ref[...]) ** 2)
        pbf = jnp.exp(_CP * (phi - op_ref[...]) ** 2)
        outs = (o0_ref, o1_ref, o2_ref)
        for t in range(NI):
            acc = (
                jnp.dot(w50_ref[t], rbf, preferred_element_type=jnp.float32)
                + jnp.dot(w6_ref[t], tbf, preferred_element_type=jnp.float32)
                + jnp.dot(w12_ref[t], pbf, preferred_element_type=jnp.float32)
                + b_ref[t]
            )
            outs[t][...] = jnp.transpose(jnp.maximum(acc, 0.0) * C)

    out = pl.pallas_call(
        body,
        grid=(nb,),
        in_specs=[
            pl.BlockSpec((1, 8, _GEO_PAD), lambda i: (i, 0, 0)),
            pl.BlockSpec((NI, HID, NG_D), lambda i: (0, 0, 0)),
            pl.BlockSpec((NI, HID, NG_T), lambda i: (0, 0, 0)),
            pl.BlockSpec((NI, HID, NG_P), lambda i: (0, 0, 0)),
            pl.BlockSpec((NI, HID, 1), lambda i: (0, 0, 0)),
            pl.BlockSpec((NG_D, 1), lambda i: (0, 0)),
            pl.BlockSpec((NG_T, 1), lambda i: (0, 0)),
            pl.BlockSpec((NG_P, 1), lambda i: (0, 0)),
        ],
        out_specs=[pl.BlockSpec((bE, HID), lambda i: (i, 0)) for _ in range(NI)],
        out_shape=[jax.ShapeDtypeStruct((E, HID), jnp.float32) for _ in range(NI)],
    )(geo, Wt50s, Wt6s, Wt12s, bfs, offd_c, offt_c, offp_c)
    return out


def _mul(hg, filt):
    bE = 8000
    nb = E // bE

    def body(a_ref, b_ref, o_ref):
        o_ref[...] = a_ref[...] * b_ref[...]

    return pl.pallas_call(
        body,
        grid=(nb,),
        in_specs=[
            pl.BlockSpec((bE, HID), lambda i: (i, 0)),
            pl.BlockSpec((bE, HID), lambda i: (i, 0)),
        ],
        out_specs=pl.BlockSpec((bE, HID), lambda i: (i, 0)),
        out_shape=jax.ShapeDtypeStruct((E, HID), jnp.float32),
    )(hg, filt)


_CD = float(-0.5 / (CUT / (NG_D - 1)) ** 2)
_CT = float(-0.5 / (np.pi / (NG_T - 1)) ** 2)
_CP = float(-0.5 / (2.0 * np.pi / (NG_P - 1)) ** 2)


def _upd(h, acc2, Wu, bu):
    bm = 5000
    nb = _RHALF // bm

    def body(h_ref, a_ref, w_ref, b_ref, o_ref):
        a = a_ref[0]
        o_ref[...] = h_ref[...] + jnp.maximum(
            jnp.dot(a, w_ref[...], preferred_element_type=jnp.float32) + b_ref[...],
            0.0,
        )

    return pl.pallas_call(
        body,
        grid=(_NC, nb),
        in_specs=[
            pl.BlockSpec((bm, HID), lambda c, i: (c * nb + i, 0)),
            pl.BlockSpec((1, bm, HID), lambda c, i: (c, i, 0)),
            pl.BlockSpec((HID, HID), lambda c, i: (0, 0)),
            pl.BlockSpec((1, HID), lambda c, i: (0, 0)),
        ],
        out_specs=pl.BlockSpec((bm, HID), lambda c, i: (c * nb + i, 0)),
        out_shape=jax.ShapeDtypeStruct((N, HID), jnp.float32),
    )(h, acc2, Wu, bu)


def _dec(pool2, Wl, bl, W1, b1, W2, b2, W3, b3, Wn1, bn1, Wn2, bn2):
    def body(p_ref, wl_ref, bl_ref, w1_ref, b1_ref, w2_ref, b2_ref, w3_ref, b3_ref,
             wn1_ref, bn1_ref, wn2_ref, bn2_ref, z_ref, nf_ref, pn_ref):
        pooled = jnp.concatenate([p_ref[0, :_BHALF], p_ref[1, :_BHALF]], axis=0)
        z = jnp.dot(pooled, wl_ref[...], preferred_element_type=jnp.float32) + bl_ref[...]
        d1 = jnp.maximum(jnp.dot(z, w1_ref[...], preferred_element_type=jnp.float32) + b1_ref[...], 0.0)
        d2 = jnp.maximum(jnp.dot(d1, w2_ref[...], preferred_element_type=jnp.float32) + b2_ref[...], 0.0)
        nf = jnp.dot(d2, w3_ref[...], preferred_element_type=jnp.float32) + b3_ref[...]
        n1 = jnp.maximum(jnp.dot(z, wn1_ref[...], preferred_element_type=jnp.float32) + bn1_ref[...], 0.0)
        pn = jnp.maximum(jnp.dot(n1, wn2_ref[...], preferred_element_type=jnp.float32) + bn2_ref[...], 0.0)
        z_ref[...] = z
        nf_ref[...] = nf
        pn_ref[...] = pn

    return pl.pallas_call(
        body,
        out_shape=(
            jax.ShapeDtypeStruct((B, LAT), jnp.float32),
            jax.ShapeDtypeStruct((B, MAXN * F), jnp.float32),
            jax.ShapeDtypeStruct((B, 1), jnp.float32),
        ),
    )(pool2, Wl, bl, W1, b1, W2, b2, W3, b3, Wn1, bn1, Wn2, bn2)


def kernel(x, pos, batch, edge_index_3rd, W_emb, b_emb, W_filt, b_filt, W_upd, b_upd,
           W_lin1, b_lin1, Wd1, bd1, Wd2, bd2, Wd3, bd3, Wn1, bn1, Wn2, bn2):
    f32 = jnp.float32
    ei = edge_index_3rd.astype(jnp.int32)
    dst = ei[0]
    src = ei[1]

    pos16 = jnp.pad(pos, ((0, 0), (0, 13)))
    geo = _sc_geo(pos16, ei)

    offd_c = jnp.asarray(np.linspace(0.0, CUT, NG_D), f32).reshape(NG_D, 1)
    offt_c = jnp.asarray(np.linspace(0.0, np.pi, NG_T), f32).reshape(NG_T, 1)
    offp_c = jnp.asarray(np.linspace(-np.pi, np.pi, NG_P), f32).reshape(NG_P, 1)
    Wt50s = jnp.transpose(W_filt[:, :NG_D, :], (0, 2, 1))
    Wt6s = jnp.transpose(W_filt[:, NG_D:NG_D + NG_T, :], (0, 2, 1))
    Wt12s = jnp.transpose(W_filt[:, NG_D + NG_T:, :], (0, 2, 1))
    bfs = b_filt.reshape(NI, HID, 1)
    filts = _geom_filt(geo, Wt50s, Wt6s, Wt12s, bfs, offd_c, offt_c, offp_c)

    h = _emb(x, W_emb, b_emb.reshape(1, HID))

    lo = jnp.where(dst < _RHALF, dst, _TRASH_E)
    hi = jnp.where(dst >= _RHALF, dst - _RHALF, _TRASH_E)
    idx2_e = jnp.stack([lo, hi]).astype(jnp.int32)
    zeros_e = jnp.zeros((_ZSL_E, HID), f32)

    for t in range(NI):
        hg = _sc_gather(h, src, chunk=1000)
        msg = _mul(hg, filts[t])
        acc2 = _sc_scatter(msg, idx2_e, zeros_e, racc=_RACC_E, chunk=200)
        h = _upd(h, acc2, W_upd[t], b_upd[t].reshape(1, HID))

    bat = batch.astype(jnp.int32)
    pad = _NPAD - N
    blo = jnp.concatenate([jnp.where(bat < _BHALF, bat, _TRASH_B),
                           jnp.full((pad,), _TRASH_B, jnp.int32)])
    bhi = jnp.concatenate([jnp.where(bat >= _BHALF, bat - _BHALF, _TRASH_B),
                           jnp.full((pad,), _TRASH_B, jnp.int32)])
    idx2_b = jnp.stack([blo, bhi]).astype(jnp.int32)
    hp = jnp.pad(h, ((0, pad), (0, 0)))
    pool2 = _sc_scatter(hp, idx2_b, jnp.zeros((_ZSL_B, HID), f32),
                        racc=_RACC_B, chunk=392)

    z, nf, pn = _dec(pool2, W_lin1, b_lin1.reshape(1, LAT),
                     Wd1, bd1.reshape(1, HID * 2), Wd2, bd2.reshape(1, HID * 4),
                     Wd3, bd3.reshape(1, MAXN * F), Wn1, bn1.reshape(1, HID),
                     Wn2, bn2.reshape(1, 1))
    return nf.reshape(B, MAXN, F), z, pn


# speedup vs baseline: 5.1261x; 1.2212x over previous
import functools

import numpy as np
import jax
import jax.numpy as jnp
from jax import lax
from jax.experimental import pallas as pl
from jax.experimental.pallas import tpu as pltpu
from jax.experimental.pallas import tpu_sc as plsc

N = 50000
E = 800000
B = 500
F = 5
HID = 64
LAT = 64
MAXN = 150
CUT = 10.0
NI = 3
NG_D, NG_T, NG_P = 50, 6, 12

_NC = 2
_NS = 16
_NW = _NC * _NS

_RACC_E = 50048
_ZSL_E = _RACC_E // _NS
_RACC_B = 512
_ZSL_B = _RACC_B // _NS
_NPAD = 50176


def _sc_gather(table, idx, chunk):
    M = idx.shape[0]
    D = table.shape[1]
    m = M // _NW
    steps = m // chunk
    mesh = plsc.VectorSubcoreMesh(core_axis_name="c", subcore_axis_name="s")

    @functools.partial(
        pl.kernel,
        mesh=mesh,
        out_type=jax.ShapeDtypeStruct((M, D), jnp.float32),
        compiler_params=pltpu.CompilerParams(use_tc_tiling_on_sc=False),
        scratch_types=[
            pltpu.VMEM((chunk,), jnp.int32),
            pltpu.VMEM((chunk, D), jnp.float32),
            pltpu.SemaphoreType.DMA,
        ],
    )
    def k(table_h, idx_h, out_h, idx_v, rows_v, sem):
        wid = lax.axis_index("s") * _NC + lax.axis_index("c")
        base = wid * m

        def body(t, carry):
            off = base + t * chunk
            pltpu.sync_copy(idx_h.at[pl.ds(off, chunk)], idx_v)
            pltpu.async_copy(table_h.at[idx_v], rows_v, sem).wait()
            pltpu.sync_copy(rows_v, out_h.at[pl.ds(off, chunk)])
            return carry

        lax.fori_loop(0, steps, body, 0)

    return k(table, idx)


_GEO_CHUNK = 1000
_GEO_PAD = 1008


def _sc_geo(pos16, ei):
    chunk = _GEO_CHUNK
    m = E // _NW
    steps = m // chunk
    ngr = _GEO_PAD // 16
    mesh = plsc.VectorSubcoreMesh(core_axis_name="c", subcore_axis_name="s")

    @functools.partial(
        pl.kernel,
        mesh=mesh,
        out_type=jax.ShapeDtypeStruct((E // chunk, 8, _GEO_PAD), jnp.float32),
        compiler_params=pltpu.CompilerParams(
            use_tc_tiling_on_sc=False, needs_layout_passes=False
        ),
        scratch_types=[
            pltpu.VMEM((4, chunk), jnp.int32),
            pltpu.VMEM((4, chunk, 16), jnp.float32),
            pltpu.VMEM((8, _GEO_PAD), jnp.float32),
            pltpu.SemaphoreType.DMA((4,)),
        ],
    )
    def k(pos_h, ei_h, out_h, idx_v, pts_v, out_v, sems):
        wid = lax.axis_index("s") * _NC + lax.axis_index("c")

        def body(t, carry):
            off = wid * m + t * chunk
            for q in range(4):
                pltpu.sync_copy(ei_h.at[q, pl.ds(off, chunk)], idx_v.at[q])
            for q in range(4):
                pltpu.make_async_copy(
                    pos_h.at[idx_v.at[q]], pts_v.at[q], sems.at[q]
                ).start()
            for q in range(4):
                pltpu.make_async_copy(
                    pos_h.at[idx_v.at[q]], pts_v.at[q], sems.at[q]
                ).wait()

            def group(g, carry2):
                e0 = g * 16
                lane = lax.iota(jnp.int32, 16)
                ids = jnp.minimum(e0 + lane, chunk - 1)
                coords = []
                for q in range(4):
                    qv = jnp.full((16,), q, jnp.int32)
                    coords.append([
                        plsc.load_gather(
                            pts_v, [qv, ids, jnp.full((16,), cc, jnp.int32)]
                        )
                        for cc in range(3)
                    ])
                b1 = [coords[1][d] - coords[0][d] for d in range(3)]
                b2 = [coords[2][d] - coords[1][d] for d in range(3)]
                b3 = [coords[3][d] - coords[2][d] for d in range(3)]

                def dot(a, b):
                    return a[0] * b[0] + a[1] * b[1] + a[2] * b[2]

                d3 = (b1[0] * (b2[1] * b3[2] - b2[2] * b3[1])
                      + b1[1] * (b2[2] * b3[0] - b2[0] * b3[2])
                      + b1[2] * (b2[0] * b3[1] - b2[1] * b3[0]))
                vals = [dot(b1, b1), dot(b1, b2), dot(b2, b2),
                        dot(b2, b3), dot(b1, b3), d3]
                for r, v in enumerate(vals):
                    out_v[r, pl.ds(e0, 16)] = v
                return carry2

            lax.fori_loop(0, ngr, group, 0)
            pltpu.sync_copy(out_v, out_h.at[wid * steps + t])
            return carry

        lax.fori_loop(0, steps, body, 0)

    return k(pos16, ei)


def _sc_scatter(rows, idx, zeros, racc, chunk):
    M = rows.shape[0]
    m = M // _NS
    steps = m // chunk
    zsl = racc // _NS
    mesh = plsc.VectorSubcoreMesh(core_axis_name="c", subcore_axis_name="s")

    @functools.partial(
        pl.kernel,
        mesh=mesh,
        out_type=jax.ShapeDtypeStruct((_NC, racc, 32), jnp.float32),
        compiler_params=pltpu.CompilerParams(use_tc_tiling_on_sc=False),
        scratch_types=[
            pltpu.VMEM((2, chunk), jnp.int32),
            pltpu.VMEM((2, chunk, 32), jnp.float32),
            pltpu.VMEM_SHARED((racc, 32), jnp.float32),
            pltpu.SemaphoreType.DMA((2,)),
            pltpu.SemaphoreType.DMA((2,)),
        ],
    )
    def k(rows_h, idx_h, z_h, out_h, idx_v, rows_v, acc_s, rsem, isem):
        c = lax.axis_index("c")
        s = lax.axis_index("s")
        pltpu.sync_copy(z_h, acc_s.at[pl.ds(s * zsl, zsl)])
        plsc.subcore_barrier()

        def issue(t):
            slot = lax.rem(t, 2)
            off = s * m + t * chunk
            pltpu.make_async_copy(
                rows_h.at[pl.ds(off, chunk), pl.ds(c * 32, 32)],
                rows_v.at[slot], rsem.at[slot]
            ).start()
            pltpu.make_async_copy(
                idx_h.at[pl.ds(off, chunk)], idx_v.at[slot], isem.at[slot]
            ).start()

        issue(0)

        def body(t, carry):
            slot = lax.rem(t, 2)

            @pl.when(t + 1 < steps)
            def _():
                issue(t + 1)

            off = s * m + t * chunk
            pltpu.make_async_copy(
                rows_h.at[pl.ds(off, chunk), pl.ds(c * 32, 32)],
                rows_v.at[slot], rsem.at[slot]
            ).wait()
            pltpu.make_async_copy(
                idx_h.at[pl.ds(off, chunk)], idx_v.at[slot], isem.at[slot]
            ).wait()
            pltpu.sync_copy(rows_v.at[slot], acc_s.at[idx_v.at[slot]], add=True)
            return carry

        lax.fori_loop(0, steps, body, 0)
        plsc.subcore_barrier()
        pltpu.sync_copy(acc_s.at[pl.ds(s * zsl, zsl)], out_h.at[c, pl.ds(s * zsl, zsl)])

    return k(rows, idx, zeros)


def _emb(x, W, b2):
    bm = 5000

    def body(x_ref, w_ref, b_ref, o_ref):
        o_ref[...] = (
            jnp.dot(x_ref[...], w_ref[...], preferred_element_type=jnp.float32)
            + b_ref[...]
        )

    return pl.pallas_call(
        body,
        grid=(N // bm,),
        in_specs=[
            pl.BlockSpec((bm, F), lambda i: (i, 0)),
            pl.BlockSpec((F, HID), lambda i: (0, 0)),
            pl.BlockSpec((1, HID), lambda i: (0, 0)),
        ],
        out_specs=pl.BlockSpec((bm, HID), lambda i: (i, 0)),
        out_shape=jax.ShapeDtypeStruct((N, HID), jnp.float32),
    )(x, W, b2)


def _geom_filt(geo, Wt50s, Wt6s, Wt12s, bfs, offd_c, offt_c, offp_c):
    bE = _GEO_CHUNK
    nb = E // bE

    def body(g_ref, w50_ref, w6_ref, w12_ref, b_ref,
             od_ref, ot_ref, op_ref, o0_ref, o1_ref, o2_ref):
        eps = 1e-8
        g = g_ref[0]
        s11 = g[0:1, :bE]
        s12 = g[1:2, :bE]
        s22 = g[2:3, :bE]
        s23 = g[3:4, :bE]
        s13 = g[4:5, :bE]
        d3 = g[5:6, :bE]
        dist = jnp.sqrt(s11 + eps)
        nu = jnp.sqrt(s11)
        nv = jnp.sqrt(s22)
        cos_t = -s12 / (nu * nv + eps)
        cos_t = jnp.clip(cos_t, -1.0 + 1e-7, 1.0 - 1e-7)
        theta = jnp.arctan2(jnp.sqrt(1.0 - cos_t * cos_t), cos_t)
        tx = (s12 * s23 - s13 * s22) + eps
        ty = -(d3 * s22) / (nv + eps)
        phi = jnp.arctan2(ty, tx)
        C = 0.5 * (jnp.cos(dist * (np.pi / CUT)) + 1.0) * (dist < CUT).astype(jnp.float32)
        rbf = jnp.exp(_CD * (dist - od_ref[...]) ** 2)
        tbf = jnp.exp(_CT * (theta - ot_ref[...]) ** 2)
        pbf = jnp.exp(_CP * (phi - op_ref[...]) ** 2)
        outs = (o0_ref, o1_ref, o2_ref)
        for t in range(NI):
            acc = (
                jnp.dot(w50_ref[t], rbf, preferred_element_type=jnp.float32)
                + jnp.dot(w6_ref[t], tbf, preferred_element_type=jnp.float32)
                + jnp.dot(w12_ref[t], pbf, preferred_element_type=jnp.float32)
                + b_ref[t]
            )
            outs[t][...] = jnp.transpose(jnp.maximum(acc, 0.0) * C)

    out = pl.pallas_call(
        body,
        grid=(nb,),
        in_specs=[
            pl.BlockSpec((1, 8, _GEO_PAD), lambda i: (i, 0, 0)),
            pl.BlockSpec((NI, HID, NG_D), lambda i: (0, 0, 0)),
            pl.BlockSpec((NI, HID, NG_T), lambda i: (0, 0, 0)),
            pl.BlockSpec((NI, HID, NG_P), lambda i: (0, 0, 0)),
            pl.BlockSpec((NI, HID, 1), lambda i: (0, 0, 0)),
            pl.BlockSpec((NG_D, 1), lambda i: (0, 0)),
            pl.BlockSpec((NG_T, 1), lambda i: (0, 0)),
            pl.BlockSpec((NG_P, 1), lambda i: (0, 0)),
        ],
        out_specs=[pl.BlockSpec((bE, HID), lambda i: (i, 0)) for _ in range(NI)],
        out_shape=[jax.ShapeDtypeStruct((E, HID), jnp.float32) for _ in range(NI)],
    )(geo, Wt50s, Wt6s, Wt12s, bfs, offd_c, offt_c, offp_c)
    return out


def _mul(hg, filt):
    bE = 8000
    nb = E // bE

    def body(a_ref, b_ref, o_ref):
        o_ref[...] = a_ref[...] * b_ref[...]

    return pl.pallas_call(
        body,
        grid=(nb,),
        in_specs=[
            pl.BlockSpec((bE, HID), lambda i: (i, 0)),
            pl.BlockSpec((bE, HID), lambda i: (i, 0)),
        ],
        out_specs=pl.BlockSpec((bE, HID), lambda i: (i, 0)),
        out_shape=jax.ShapeDtypeStruct((E, HID), jnp.float32),
    )(hg, filt)


_CD = float(-0.5 / (CUT / (NG_D - 1)) ** 2)
_CT = float(-0.5 / (np.pi / (NG_T - 1)) ** 2)
_CP = float(-0.5 / (2.0 * np.pi / (NG_P - 1)) ** 2)


def _upd(h, acc2, Wu, bu):
    bm = 5000
    nb = N // bm

    def body(h_ref, a0_ref, a1_ref, w_ref, b_ref, o_ref):
        a = jnp.concatenate([a0_ref[0], a1_ref[0]], axis=1)
        o_ref[...] = h_ref[...] + jnp.maximum(
            jnp.dot(a, w_ref[...], preferred_element_type=jnp.float32) + b_ref[...],
            0.0,
        )

    return pl.pallas_call(
        body,
        grid=(nb,),
        in_specs=[
            pl.BlockSpec((bm, HID), lambda i: (i, 0)),
            pl.BlockSpec((1, bm, 32), lambda i: (0, i, 0)),
            pl.BlockSpec((1, bm, 32), lambda i: (1, i, 0)),
            pl.BlockSpec((HID, HID), lambda i: (0, 0)),
            pl.BlockSpec((1, HID), lambda i: (0, 0)),
        ],
        out_specs=pl.BlockSpec((bm, HID), lambda i: (i, 0)),
        out_shape=jax.ShapeDtypeStruct((N, HID), jnp.float32),
    )(h, acc2, acc2, Wu, bu)


def _dec(pool2, Wl, bl, W1, b1, W2, b2, W3, b3, Wn1, bn1, Wn2, bn2):
    def body(p_ref, wl_ref, bl_ref, w1_ref, b1_ref, w2_ref, b2_ref, w3_ref, b3_ref,
             wn1_ref, bn1_ref, wn2_ref, bn2_ref, z_ref, nf_ref, pn_ref):
        pooled = jnp.concatenate([p_ref[0, :B], p_ref[1, :B]], axis=1)
        z = jnp.dot(pooled, wl_ref[...], preferred_element_type=jnp.float32) + bl_ref[...]
        d1 = jnp.maximum(jnp.dot(z, w1_ref[...], preferred_element_type=jnp.float32) + b1_ref[...], 0.0)
        d2 = jnp.maximum(jnp.dot(d1, w2_ref[...], preferred_element_type=jnp.float32) + b2_ref[...], 0.0)
        nf = jnp.dot(d2, w3_ref[...], preferred_element_type=jnp.float32) + b3_ref[...]
        n1 = jnp.maximum(jnp.dot(z, wn1_ref[...], preferred_element_type=jnp.float32) + bn1_ref[...], 0.0)
        pn = jnp.maximum(jnp.dot(n1, wn2_ref[...], preferred_element_type=jnp.float32) + bn2_ref[...], 0.0)
        z_ref[...] = z
        nf_ref[...] = nf
        pn_ref[...] = pn

    return pl.pallas_call(
        body,
        out_shape=(
            jax.ShapeDtypeStruct((B, LAT), jnp.float32),
            jax.ShapeDtypeStruct((B, MAXN * F), jnp.float32),
            jax.ShapeDtypeStruct((B, 1), jnp.float32),
        ),
    )(pool2, Wl, bl, W1, b1, W2, b2, W3, b3, Wn1, bn1, Wn2, bn2)


def kernel(x, pos, batch, edge_index_3rd, W_emb, b_emb, W_filt, b_filt, W_upd, b_upd,
           W_lin1, b_lin1, Wd1, bd1, Wd2, bd2, Wd3, bd3, Wn1, bn1, Wn2, bn2):
    f32 = jnp.float32
    ei = edge_index_3rd.astype(jnp.int32)
    dst = ei[0]
    src = ei[1]

    pos16 = jnp.pad(pos, ((0, 0), (0, 13)))
    geo = _sc_geo(pos16, ei)

    offd_c = jnp.asarray(np.linspace(0.0, CUT, NG_D), f32).reshape(NG_D, 1)
    offt_c = jnp.asarray(np.linspace(0.0, np.pi, NG_T), f32).reshape(NG_T, 1)
    offp_c = jnp.asarray(np.linspace(-np.pi, np.pi, NG_P), f32).reshape(NG_P, 1)
    Wt50s = jnp.transpose(W_filt[:, :NG_D, :], (0, 2, 1))
    Wt6s = jnp.transpose(W_filt[:, NG_D:NG_D + NG_T, :], (0, 2, 1))
    Wt12s = jnp.transpose(W_filt[:, NG_D + NG_T:, :], (0, 2, 1))
    bfs = b_filt.reshape(NI, HID, 1)
    filts = _geom_filt(geo, Wt50s, Wt6s, Wt12s, bfs, offd_c, offt_c, offp_c)

    h = _emb(x, W_emb, b_emb.reshape(1, HID))

    zeros_e = jnp.zeros((_ZSL_E, 32), f32)

    for t in range(NI):
        hg = _sc_gather(h, src, chunk=1000)
        msg = _mul(hg, filts[t])
        acc2 = _sc_scatter(msg, dst, zeros_e, racc=_RACC_E, chunk=400)
        h = _upd(h, acc2, W_upd[t], b_upd[t].reshape(1, HID))

    bat = batch.astype(jnp.int32)
    pad = _NPAD - N
    idx_b = jnp.pad(bat, (0, pad))
    hp = jnp.pad(h, ((0, pad), (0, 0)))
    pool2 = _sc_scatter(hp, idx_b, jnp.zeros((_ZSL_B, 32), f32),
                        racc=_RACC_B, chunk=392)

    z, nf, pn = _dec(pool2, W_lin1, b_lin1.reshape(1, LAT),
                     Wd1, bd1.reshape(1, HID * 2), Wd2, bd2.reshape(1, HID * 4),
                     Wd3, bd3.reshape(1, MAXN * F), Wn1, bn1.reshape(1, HID),
                     Wn2, bn2.reshape(1, 1))
    return nf.reshape(B, MAXN, F), z, pn
